# baseline (speedup 1.0000x reference)
import functools
import jax
import jax.numpy as jnp
from jax import lax
from jax.experimental import pallas as pl
from jax.experimental.pallas import tpu as pltpu
from jax.experimental.pallas import tpu_sc as plsc

N_TOK = 4096
D_MODEL = 2048
N_EXP = 8
TOPK = 2

NC = 2
NT = 16
L = 16
BLK = 256
M = N_TOK * TOPK
NW = NC * NT
A = M // NW
M_PAD = M + N_EXP * BLK
NB = M_PAD // BLK
ACH = A // L


def _gating_body(x_ref, gw_ref, gb_ref, idx_ref, w_ref):
    x = x_ref[...]
    logits = jax.lax.dot_general(
        x, gw_ref[...], (((1,), (1,)), ((), ())),
        preferred_element_type=jnp.float32) + gb_ref[...]
    e_iota = jax.lax.broadcasted_iota(jnp.int32, logits.shape, 1)
    m1 = jnp.max(logits, axis=1, keepdims=True)
    i1 = jnp.min(jnp.where(logits == m1, e_iota, N_EXP), axis=1, keepdims=True)
    l2 = jnp.where(e_iota == i1, -jnp.inf, logits)
    m2 = jnp.max(l2, axis=1, keepdims=True)
    i2 = jnp.min(jnp.where(l2 == m2, e_iota, N_EXP), axis=1, keepdims=True)
    w0 = 1.0 / (1.0 + jnp.exp(m2 - m1))
    w1 = 1.0 - w0
    idx_ref[...] = jnp.concatenate([i1, i2], axis=1)
    w_ref[...] = jnp.concatenate([w0, w1], axis=1)


def _gating(x, gate_w, gate_b):
    blk = 1024
    return pl.pallas_call(
        _gating_body,
        grid=(N_TOK // blk,),
        in_specs=[
            pl.BlockSpec((blk, D_MODEL), lambda t: (t, 0)),
            pl.BlockSpec((N_EXP, D_MODEL), lambda t: (0, 0)),
            pl.BlockSpec((1, N_EXP), lambda t: (0, 0)),
        ],
        out_specs=[
            pl.BlockSpec((blk, TOPK), lambda t: (t, 0)),
            pl.BlockSpec((blk, TOPK), lambda t: (t, 0)),
        ],
        out_shape=[
            jax.ShapeDtypeStruct((N_TOK, TOPK), jnp.int32),
            jax.ShapeDtypeStruct((N_TOK, TOPK), jnp.float32),
        ],
    )(x, gate_w, gate_b.reshape(1, N_EXP))


def _lane_scalar(v, e):
    lane = lax.iota(jnp.int32, L)
    return jnp.sum(jnp.where(lane == e, v, 0))


def _dispatch_body(top_hbm, x_hbm, xg_hbm, pos_hbm, be_hbm,
                   ef_v, lrank_v, allc_v, start_v, gpos_v, tok_v,
                   rows_v, be_v, sem, sem2):
    c = lax.axis_index("c")
    t = lax.axis_index("s")
    tile_lin = c * NT + t
    base_a = tile_lin * A

    pltpu.sync_copy(top_hbm, ef_v)

    lane = lax.iota(jnp.int32, L)

    for tt in range(NW):
        allc_v[pl.ds(tt * L, L)] = jnp.zeros((L,), jnp.int32)

    def hist_body(ch, _):
        efc = ef_v[pl.ds(ch * L, L)]
        h = jnp.zeros((L,), jnp.int32)
        for e in range(N_EXP):
            ce = jnp.sum(jnp.where(efc == e, 1, 0))
            h = h + jnp.where(lane == e, ce, 0)
        tt = ch // ACH
        allc_v[pl.ds(tt * L, L)] = allc_v[pl.ds(tt * L, L)] + h
        return 0

    lax.fori_loop(0, M // L, hist_body, 0)

    for e in range(N_EXP):
        carry = jnp.int32(0)
        for ch in range(ACH):
            efc = ef_v[pl.ds(base_a + ch * L, L)]
            m = efc == e
            ones = jnp.where(m, 1, 0).astype(jnp.int32)
            cs = plsc.cumsum(ones)
            lr = cs - 1 + carry
            cur = lrank_v[pl.ds(ch * L, L)]
            lrank_v[pl.ds(ch * L, L)] = jnp.where(m, lr, cur)
            carry = carry + cs[L - 1]

    total = jnp.zeros((L,), jnp.int32)
    prefix = jnp.zeros((L,), jnp.int32)
    for tt in range(NW):
        v = allc_v[pl.ds(tt * L, L)]
        total = total + v
        prefix = prefix + jnp.where(jnp.int32(tt) < tile_lin, v, 0)

    padded = ((total + (BLK - 1)) // BLK) * BLK
    eb_end = plsc.cumsum(padded)
    eb = eb_end - padded
    start_v[...] = eb + prefix

    for ch in range(ACH):
        efc = ef_v[pl.ds(base_a + ch * L, L)]
        s = plsc.load_gather(start_v, [efc])
        lr = lrank_v[pl.ds(ch * L, L)]
        gpos_v[ch] = s + lr
        tok_v[ch] = (lax.iota(jnp.int32, L) + (tile_lin * A + ch * L)) >> 1
    pltpu.sync_copy(gpos_v, pos_hbm.at[pl.ds(tile_lin * ACH, ACH), :])

    nbuf = 3
    cpg_next = pltpu.async_copy(x_hbm.at[tok_v.at[0]], rows_v.at[0], sem)
    prev_sc = None
    for ch in range(ACH):
        cpg_next.wait()
        if prev_sc is not None:
            prev_sc.wait()
        if ch + 1 < ACH:
            cpg_next = pltpu.async_copy(
                x_hbm.at[tok_v.at[ch + 1]], rows_v.at[(ch + 1) % nbuf], sem)
        prev_sc = pltpu.async_copy(
            rows_v.at[ch % nbuf], xg_hbm.at[gpos_v.at[ch]], sem2)
    prev_sc.wait()

    @pl.when(tile_lin == 0)
    def _():
        nchunk = (NB + L - 1) // L
        for cc in range(nchunk):
            bstart = (lax.iota(jnp.int32, L) + cc * L) * BLK
            acc = jnp.zeros((L,), jnp.int32)
            for e in range(N_EXP):
                ee = _lane_scalar(eb_end, e)
                acc = acc + jnp.where(bstart >= ee, 1, 0)
            be_v[pl.ds(cc * L, L)] = acc
        pltpu.sync_copy(be_v.at[pl.ds(0, NB)], be_hbm)


def _dispatch(top_flat, x):
    mesh = plsc.VectorSubcoreMesh(
        core_axis_name="c", subcore_axis_name="s",
        num_cores=NC, num_subcores=NT)
    f = pl.kernel(
        _dispatch_body,
        out_type=[
            jax.ShapeDtypeStruct((M_PAD, D_MODEL), jnp.float32),
            jax.ShapeDtypeStruct((M // L, L), jnp.int32),
            jax.ShapeDtypeStruct((NB,), jnp.int32),
        ],
        mesh=mesh,
        compiler_params=pltpu.CompilerParams(needs_layout_passes=False),
        scratch_types=[
            pltpu.VMEM((M,), jnp.int32),
            pltpu.VMEM((A,), jnp.int32),
            pltpu.VMEM((NW * L,), jnp.int32),
            pltpu.VMEM((L,), jnp.int32),
            pltpu.VMEM((ACH, L), jnp.int32),
            pltpu.VMEM((ACH, L), jnp.int32),
            pltpu.VMEM((3, L, D_MODEL), jnp.float32),
            pltpu.VMEM((48,), jnp.int32),
            pltpu.SemaphoreType.DMA,
            pltpu.SemaphoreType.DMA,
        ],
    )
    return f(top_flat, x)


def _gemm_body(be_ref, xg_ref, w_ref, b_ref, out_ref):
    b = pl.program_id(0)

    @pl.when(be_ref[b] < N_EXP)
    def _():
        y = jax.lax.dot_general(
            xg_ref[...], w_ref[0], (((1,), (1,)), ((), ())),
            preferred_element_type=jnp.float32)
        out_ref[...] = y + b_ref[0]


def _gemm(be, xg, expert_w, expert_b):
    def wmap(b, be_ref):
        return (jnp.minimum(be_ref[b], N_EXP - 1), 0, 0)

    grid_spec = pltpu.PrefetchScalarGridSpec(
        num_scalar_prefetch=1,
        grid=(NB,),
        in_specs=[
            pl.BlockSpec((BLK, D_MODEL),
                         lambda b, be_ref: (jnp.where(be_ref[b] < N_EXP, b, 0), 0)),
            pl.BlockSpec((1, D_MODEL, D_MODEL), wmap),
            pl.BlockSpec((1, 1, D_MODEL), wmap),
        ],
        out_specs=pl.BlockSpec((BLK, D_MODEL), lambda b, be_ref: (b, 0)),
    )
    return pl.pallas_call(
        _gemm_body,
        grid_spec=grid_spec,
        out_shape=jax.ShapeDtypeStruct((M_PAD, D_MODEL), jnp.float32),
        compiler_params=pltpu.CompilerParams(
            vmem_limit_bytes=100 * 1024 * 1024),
    )(be, xg, expert_w, expert_b.reshape(N_EXP, 1, D_MODEL))


TOK_T = N_TOK // (NC * NT)
TCH = TOK_T // 8


def _combine_body(yg_hbm, pos_hbm, w_hbm, out_hbm,
                  pos_v, wv_v, rows_v, out_v, sem, sem2):
    c = lax.axis_index("c")
    t = lax.axis_index("s")
    tile_lin = c * NT + t
    base_tok = tile_lin * TOK_T

    pltpu.sync_copy(pos_hbm.at[pl.ds(tile_lin * ACH, ACH), :], pos_v)
    pltpu.sync_copy(w_hbm.at[pl.ds(base_tok * TOPK, TOK_T * TOPK)], wv_v)

    cpg_next = pltpu.async_copy(yg_hbm.at[pos_v.at[0]], rows_v.at[0], sem)
    prev_out = None
    for ch in range(TCH):
        cpg_next.wait()
        if ch + 1 < TCH:
            cpg_next = pltpu.async_copy(
                yg_hbm.at[pos_v.at[ch + 1]], rows_v.at[(ch + 1) % 2], sem)
        wvc = wv_v[pl.ds(ch * L, L)]
        for i in range(8):
            w0 = wvc[2 * i]
            w1 = wvc[2 * i + 1]

            def body(j, _):
                for u in range(2):
                    o = (2 * j + u) * L
                    r0 = rows_v[ch % 2, 2 * i, pl.ds(o, L)]
                    r1 = rows_v[ch % 2, 2 * i + 1, pl.ds(o, L)]
                    out_v[ch % 2, i, pl.ds(o, L)] = w0 * r0 + w1 * r1
                return 0

            lax.fori_loop(0, D_MODEL // (2 * L), body, 0)
        if prev_out is not None:
            prev_out.wait()
        prev_out = pltpu.async_copy(
            out_v.at[ch % 2], out_hbm.at[pl.ds(base_tok + ch * 8, 8), :],
            sem2)
    prev_out.wait()


def _combine(yg, pos, w_flat):
    mesh = plsc.VectorSubcoreMesh(
        core_axis_name="c", subcore_axis_name="s",
        num_cores=NC, num_subcores=NT)
    f = pl.kernel(
        _combine_body,
        out_type=jax.ShapeDtypeStruct((N_TOK, D_MODEL), jnp.float32),
        mesh=mesh,
        compiler_params=pltpu.CompilerParams(needs_layout_passes=False),
        scratch_types=[
            pltpu.VMEM((ACH, L), jnp.int32),
            pltpu.VMEM((A,), jnp.float32),
            pltpu.VMEM((2, L, D_MODEL), jnp.float32),
            pltpu.VMEM((2, 8, D_MODEL), jnp.float32),
            pltpu.SemaphoreType.DMA,
            pltpu.SemaphoreType.DMA,
        ],
    )
    return f(yg, pos, w_flat)


def kernel(x, gate_w, gate_b, expert_w, expert_b):
    top_idx, w = _gating(x, gate_w, gate_b)
    xg, pos, be = _dispatch(top_idx.reshape(-1), x)
    yg = _gemm(be, xg, expert_w, expert_b)
    final = _combine(yg, pos, w.reshape(-1))
    return final, top_idx


# speedup vs baseline: 1.1318x; 1.1318x over previous
import functools
import jax
import jax.numpy as jnp
from jax import lax
from jax.experimental import pallas as pl
from jax.experimental.pallas import tpu as pltpu
from jax.experimental.pallas import tpu_sc as plsc

N_TOK = 4096
D_MODEL = 2048
N_EXP = 8
TOPK = 2

NC = 2
NT = 16
L = 16
BLK = 256
M = N_TOK * TOPK
NW = NC * NT
A = M // NW
M_PAD = M + N_EXP * BLK
NB = M_PAD // BLK
ACH = A // L


def _gating_body(x_ref, gw_ref, gb_ref, idx_ref, w_ref):
    x = x_ref[...]
    logits = jax.lax.dot_general(
        x, gw_ref[...], (((1,), (1,)), ((), ())),
        preferred_element_type=jnp.float32) + gb_ref[...]
    e_iota = jax.lax.broadcasted_iota(jnp.int32, logits.shape, 1)
    m1 = jnp.max(logits, axis=1, keepdims=True)
    i1 = jnp.min(jnp.where(logits == m1, e_iota, N_EXP), axis=1, keepdims=True)
    l2 = jnp.where(e_iota == i1, -jnp.inf, logits)
    m2 = jnp.max(l2, axis=1, keepdims=True)
    i2 = jnp.min(jnp.where(l2 == m2, e_iota, N_EXP), axis=1, keepdims=True)
    w0 = 1.0 / (1.0 + jnp.exp(m2 - m1))
    w1 = 1.0 - w0
    idx_ref[...] = jnp.concatenate([i1, i2], axis=1)
    w_ref[...] = jnp.concatenate([w0, w1], axis=1)


def _gating(x, gate_w, gate_b):
    blk = 1024
    return pl.pallas_call(
        _gating_body,
        grid=(N_TOK // blk,),
        in_specs=[
            pl.BlockSpec((blk, D_MODEL), lambda t: (t, 0)),
            pl.BlockSpec((N_EXP, D_MODEL), lambda t: (0, 0)),
            pl.BlockSpec((1, N_EXP), lambda t: (0, 0)),
        ],
        out_specs=[
            pl.BlockSpec((blk, TOPK), lambda t: (t, 0)),
            pl.BlockSpec((blk, TOPK), lambda t: (t, 0)),
        ],
        out_shape=[
            jax.ShapeDtypeStruct((N_TOK, TOPK), jnp.int32),
            jax.ShapeDtypeStruct((N_TOK, TOPK), jnp.float32),
        ],
    )(x, gate_w, gate_b.reshape(1, N_EXP))


def _lane_scalar(v, e):
    lane = lax.iota(jnp.int32, L)
    return jnp.sum(jnp.where(lane == e, v, 0))


def _dispatch_body(top_hbm, x_hbm, xg_hbm, pos_hbm, be_hbm,
                   ef_v, lrank_v, allc_v, start_v, gpos_v, tok_v,
                   rows_v, be_v, sem, sem2):
    c = lax.axis_index("c")
    t = lax.axis_index("s")
    tile_lin = c * NT + t
    base_a = tile_lin * A

    pltpu.sync_copy(top_hbm, ef_v)

    lane = lax.iota(jnp.int32, L)

    for tt in range(NW):
        allc_v[pl.ds(tt * L, L)] = jnp.zeros((L,), jnp.int32)

    def hist_body(ch, _):
        efc = ef_v[pl.ds(ch * L, L)]
        h = jnp.zeros((L,), jnp.int32)
        for e in range(N_EXP):
            ce = jnp.sum(jnp.where(efc == e, 1, 0))
            h = h + jnp.where(lane == e, ce, 0)
        tt = ch // ACH
        allc_v[pl.ds(tt * L, L)] = allc_v[pl.ds(tt * L, L)] + h
        return 0

    lax.fori_loop(0, M // L, hist_body, 0)

    for e in range(N_EXP):
        carry = jnp.int32(0)
        for ch in range(ACH):
            efc = ef_v[pl.ds(base_a + ch * L, L)]
            m = efc == e
            ones = jnp.where(m, 1, 0).astype(jnp.int32)
            cs = plsc.cumsum(ones)
            lr = cs - 1 + carry
            cur = lrank_v[pl.ds(ch * L, L)]
            lrank_v[pl.ds(ch * L, L)] = jnp.where(m, lr, cur)
            carry = carry + cs[L - 1]

    total = jnp.zeros((L,), jnp.int32)
    prefix = jnp.zeros((L,), jnp.int32)
    for tt in range(NW):
        v = allc_v[pl.ds(tt * L, L)]
        total = total + v
        prefix = prefix + jnp.where(jnp.int32(tt) < tile_lin, v, 0)

    padded = ((total + (BLK - 1)) // BLK) * BLK
    eb_end = plsc.cumsum(padded)
    eb = eb_end - padded
    start_v[...] = eb + prefix

    for ch in range(ACH):
        efc = ef_v[pl.ds(base_a + ch * L, L)]
        s = plsc.load_gather(start_v, [efc])
        lr = lrank_v[pl.ds(ch * L, L)]
        gpos_v[ch] = s + lr
        tok_v[ch] = (lax.iota(jnp.int32, L) + (tile_lin * A + ch * L)) >> 1
    pltpu.sync_copy(gpos_v, pos_hbm.at[pl.ds(tile_lin * ACH, ACH), :])

    nbuf = 3
    cpg_next = pltpu.async_copy(x_hbm.at[tok_v.at[0]], rows_v.at[0], sem)
    prev_sc = None
    for ch in range(ACH):
        cpg_next.wait()
        if prev_sc is not None:
            prev_sc.wait()
        if ch + 1 < ACH:
            cpg_next = pltpu.async_copy(
                x_hbm.at[tok_v.at[ch + 1]], rows_v.at[(ch + 1) % nbuf], sem)
        prev_sc = pltpu.async_copy(
            rows_v.at[ch % nbuf], xg_hbm.at[gpos_v.at[ch]], sem2)
    prev_sc.wait()

    @pl.when(tile_lin == 0)
    def _():
        nchunk = (NB + L - 1) // L
        for cc in range(nchunk):
            bstart = (lax.iota(jnp.int32, L) + cc * L) * BLK
            acc = jnp.zeros((L,), jnp.int32)
            for e in range(N_EXP):
                ee = _lane_scalar(eb_end, e)
                acc = acc + jnp.where(bstart >= ee, 1, 0)
            be_v[pl.ds(cc * L, L)] = acc
        pltpu.sync_copy(be_v.at[pl.ds(0, NB)], be_hbm)


def _dispatch(top_flat, x):
    mesh = plsc.VectorSubcoreMesh(
        core_axis_name="c", subcore_axis_name="s",
        num_cores=NC, num_subcores=NT)
    f = pl.kernel(
        _dispatch_body,
        out_type=[
            jax.ShapeDtypeStruct((M_PAD, D_MODEL), jnp.float32),
            jax.ShapeDtypeStruct((M // L, L), jnp.int32),
            jax.ShapeDtypeStruct((NB,), jnp.int32),
        ],
        mesh=mesh,
        compiler_params=pltpu.CompilerParams(needs_layout_passes=False),
        scratch_types=[
            pltpu.VMEM((M,), jnp.int32),
            pltpu.VMEM((A,), jnp.int32),
            pltpu.VMEM((NW * L,), jnp.int32),
            pltpu.VMEM((L,), jnp.int32),
            pltpu.VMEM((ACH, L), jnp.int32),
            pltpu.VMEM((ACH, L), jnp.int32),
            pltpu.VMEM((3, L, D_MODEL), jnp.float32),
            pltpu.VMEM((48,), jnp.int32),
            pltpu.SemaphoreType.DMA,
            pltpu.SemaphoreType.DMA,
        ],
    )
    return f(top_flat, x)


def _gemm_body(be_ref, xg_ref, w_ref, b_ref, out_ref):
    b = pl.program_id(0)

    @pl.when(be_ref[b] < N_EXP)
    def _():
        y = jax.lax.dot_general(
            xg_ref[...], w_ref[0], (((1,), (1,)), ((), ())),
            preferred_element_type=jnp.float32)
        out_ref[...] = y + b_ref[0]


def _gemm(be, xg, expert_w, expert_b):
    def wmap(b, be_ref):
        return (jnp.minimum(be_ref[b], N_EXP - 1), 0, 0)

    grid_spec = pltpu.PrefetchScalarGridSpec(
        num_scalar_prefetch=1,
        grid=(NB,),
        in_specs=[
            pl.BlockSpec((BLK, D_MODEL),
                         lambda b, be_ref: (jnp.where(be_ref[b] < N_EXP, b, 0), 0)),
            pl.BlockSpec((1, D_MODEL, D_MODEL), wmap),
            pl.BlockSpec((1, 1, D_MODEL), wmap),
        ],
        out_specs=pl.BlockSpec((BLK, D_MODEL), lambda b, be_ref: (b, 0)),
    )
    return pl.pallas_call(
        _gemm_body,
        grid_spec=grid_spec,
        out_shape=jax.ShapeDtypeStruct((M_PAD, D_MODEL), jnp.float32),
        compiler_params=pltpu.CompilerParams(
            vmem_limit_bytes=100 * 1024 * 1024),
    )(be, xg, expert_w, expert_b.reshape(N_EXP, 1, D_MODEL))


TOK_T = N_TOK // (NC * NT)
TCH = TOK_T // 8


def _combine_body(yg_hbm, pos_hbm, w_hbm, out_hbm,
                  pos_v, wv_v, rows_v, out_v, sem, sem2):
    c = lax.axis_index("c")
    t = lax.axis_index("s")
    tile_lin = c * NT + t
    base_tok = tile_lin * TOK_T

    pltpu.sync_copy(pos_hbm.at[pl.ds(tile_lin * ACH, ACH), :], pos_v)
    pltpu.sync_copy(w_hbm.at[pl.ds(base_tok * TOPK, TOK_T * TOPK)], wv_v)

    cpg_next = pltpu.async_copy(yg_hbm.at[pos_v.at[0]], rows_v.at[0], sem)
    prev_out = None
    for ch in range(TCH):
        cpg_next.wait()
        if ch + 1 < TCH:
            cpg_next = pltpu.async_copy(
                yg_hbm.at[pos_v.at[ch + 1]], rows_v.at[(ch + 1) % 2], sem)
        wvc = wv_v[pl.ds(ch * L, L)]
        for i in range(8):
            w0 = wvc[2 * i]
            w1 = wvc[2 * i + 1]

            def body(j, _):
                r0 = rows_v[ch % 2, 2 * i, pl.ds(j * L, L)]
                r1 = rows_v[ch % 2, 2 * i + 1, pl.ds(j * L, L)]
                out_v[ch % 2, i, pl.ds(j * L, L)] = w0 * r0 + w1 * r1
                return 0

            lax.fori_loop(0, D_MODEL // L, body, 0)
        if prev_out is not None:
            prev_out.wait()
        prev_out = pltpu.async_copy(
            out_v.at[ch % 2], out_hbm.at[pl.ds(base_tok + ch * 8, 8), :],
            sem2)
    prev_out.wait()


def _combine(yg, pos, w_flat):
    mesh = plsc.VectorSubcoreMesh(
        core_axis_name="c", subcore_axis_name="s",
        num_cores=NC, num_subcores=NT)
    f = pl.kernel(
        _combine_body,
        out_type=jax.ShapeDtypeStruct((N_TOK, D_MODEL), jnp.float32),
        mesh=mesh,
        compiler_params=pltpu.CompilerParams(needs_layout_passes=False),
        scratch_types=[
            pltpu.VMEM((ACH, L), jnp.int32),
            pltpu.VMEM((A,), jnp.float32),
            pltpu.VMEM((2, L, D_MODEL), jnp.float32),
            pltpu.VMEM((2, 8, D_MODEL), jnp.float32),
            pltpu.SemaphoreType.DMA,
            pltpu.SemaphoreType.DMA,
        ],
    )
    return f(yg, pos, w_flat)


def kernel(x, gate_w, gate_b, expert_w, expert_b):
    top_idx, w = _gating(x, gate_w, gate_b)
    xg, pos, be = _dispatch(top_idx.reshape(-1), x)
    yg = _gemm(be, xg, expert_w, expert_b)
    final = _combine(yg, pos, w.reshape(-1))
    return final, top_idx
